# baseline (speedup 1.0000x reference)
import functools

import jax
import jax.numpy as jnp
from jax.experimental import pallas as pl
from jax.experimental.pallas import tpu as pltpu
from jax.experimental.pallas import tpu_sc as plsc

BATCH = 16384
EMBED_DIM = 64
HIDDEN_DIM = 128
OUTPUT_DIM = 64

_NUM_CORES = 2
_NUM_SUBCORES = 16
_NUM_WORKERS = _NUM_CORES * _NUM_SUBCORES
_B_PER_WORKER = BATCH // _NUM_WORKERS
_WIDE = 2 * EMBED_DIM


def _sc_gather_wide(table_wide, idx_half):
    mesh = plsc.VectorSubcoreMesh(core_axis_name="c", subcore_axis_name="s")

    @functools.partial(
        pl.kernel,
        mesh=mesh,
        out_type=jax.ShapeDtypeStruct((BATCH, _WIDE), table_wide.dtype),
        scratch_types=[
            pltpu.VMEM((_B_PER_WORKER,), jnp.int32),
            pltpu.VMEM((_B_PER_WORKER, _WIDE), jnp.float32),
            pltpu.SemaphoreType.DMA,
        ],
    )
    def gather_kernel(table_hbm, idx_hbm, out_hbm, idx_v, rows_v, sem):
        wid = jax.lax.axis_index("s") * _NUM_CORES + jax.lax.axis_index("c")
        base = wid * _B_PER_WORKER
        pltpu.sync_copy(idx_hbm.at[pl.ds(base, _B_PER_WORKER)], idx_v)
        pltpu.async_copy(table_hbm.at[idx_v], rows_v, sem).wait()
        pltpu.sync_copy(rows_v, out_hbm.at[pl.ds(base, _B_PER_WORKER)])

    return gather_kernel(table_wide, idx_half)


_MLP_BLOCK = 2048


def _mlp_kernel(x_ref, par_ref, w1_ref, b1_ref, w2_ref, b2_ref, o_ref):
    x = x_ref[...]
    odd = par_ref[...] > 0
    emb = jnp.where(odd, x[:, EMBED_DIM:], x[:, :EMBED_DIM])
    h = jnp.dot(emb, w1_ref[...], preferred_element_type=jnp.float32)
    h = jnp.maximum(h + b1_ref[...], 0.0)
    o_ref[...] = (
        jnp.dot(h, w2_ref[...], preferred_element_type=jnp.float32) + b2_ref[...]
    )


def _tc_mlp(x_wide, parity, W1, b1, W2, b2):
    grid = (BATCH // _MLP_BLOCK,)
    return pl.pallas_call(
        _mlp_kernel,
        grid=grid,
        in_specs=[
            pl.BlockSpec((_MLP_BLOCK, _WIDE), lambda i: (i, 0)),
            pl.BlockSpec((_MLP_BLOCK, 1), lambda i: (i, 0)),
            pl.BlockSpec((EMBED_DIM, HIDDEN_DIM), lambda i: (0, 0)),
            pl.BlockSpec((1, HIDDEN_DIM), lambda i: (0, 0)),
            pl.BlockSpec((HIDDEN_DIM, OUTPUT_DIM), lambda i: (0, 0)),
            pl.BlockSpec((1, OUTPUT_DIM), lambda i: (0, 0)),
        ],
        out_specs=pl.BlockSpec((_MLP_BLOCK, OUTPUT_DIM), lambda i: (i, 0)),
        out_shape=jax.ShapeDtypeStruct((BATCH, OUTPUT_DIM), jnp.float32),
    )(
        x_wide,
        parity,
        W1,
        b1.reshape(1, HIDDEN_DIM),
        W2,
        b2.reshape(1, OUTPUT_DIM),
    )


def kernel(llm_ids, emb_table, W1, b1, W2, b2):
    ids = llm_ids.astype(jnp.int32)
    table_wide = emb_table.reshape(emb_table.shape[0] // 2, _WIDE)
    wide = _sc_gather_wide(table_wide, ids >> 1)
    parity = (ids & 1).reshape(BATCH, 1)
    return _tc_mlp(wide, parity, W1, b1, W2, b2)
